# baseline (speedup 1.0000x reference)
import numpy as np
import jax
import jax.numpy as jnp
from jax.experimental import pallas as pl
from jax.experimental.pallas import tpu as pltpu

_RC = np.arange(9)
_J = np.arange(24)
_ROT_PERM = (_J[None, :] * 9 + _RC[:, None]).reshape(-1)


def _head_kernel(x_ref, wr_ref, wb_ref, wc_ref, rot_ref, betas_ref, cam_ref):
    blk = x_ref.shape[2]
    for b in range(blk):
        xb = x_ref[0, :, b, :]
        rot_ref[0, b] = jax.lax.dot_general(
            wr_ref[...], xb, (((1,), (1,)), ((), ())),
            preferred_element_type=jnp.float32)
        betas_ref[0, :, b, :] = jax.lax.dot_general(
            wb_ref[...], xb, (((1,), (1,)), ((), ())),
            preferred_element_type=jnp.float32)
        cam_ref[0, :, b, :] = jax.lax.dot_general(
            wc_ref[...], xb, (((1,), (1,)), ((), ())),
            preferred_element_type=jnp.float32)


def kernel(x, pred_class, W_rot, W_betas, W_cam):
    stage, bs, nq, ch = x.shape

    xv = jnp.transpose(x, (0, 2, 1, 3))
    wr_t = W_rot.T[_ROT_PERM]
    wb_t = W_betas.T
    wc_t = W_cam.T

    blk = 16
    grid = (stage, bs // blk)

    rot, betas, cam = pl.pallas_call(
        _head_kernel,
        grid=grid,
        in_specs=[
            pl.BlockSpec((1, nq, blk, ch), lambda i, j: (i, 0, j, 0)),
            pl.BlockSpec((216, ch), lambda i, j: (0, 0)),
            pl.BlockSpec((10, ch), lambda i, j: (0, 0)),
            pl.BlockSpec((3, ch), lambda i, j: (0, 0)),
        ],
        out_specs=[
            pl.BlockSpec((1, blk, 216, nq), lambda i, j: (i, j, 0, 0)),
            pl.BlockSpec((1, 10, blk, nq), lambda i, j: (i, 0, j, 0)),
            pl.BlockSpec((1, 3, blk, nq), lambda i, j: (i, 0, j, 0)),
        ],
        out_shape=[
            jax.ShapeDtypeStruct((stage, bs, 216, nq), jnp.float32),
            jax.ShapeDtypeStruct((stage, 10, bs, nq), jnp.float32),
            jax.ShapeDtypeStruct((stage, 3, bs, nq), jnp.float32),
        ],
        compiler_params=pltpu.CompilerParams(
            dimension_semantics=("parallel", "parallel")),
    )(xv, wr_t, wb_t, wc_t)

    rotmat = jnp.transpose(
        rot.reshape(stage, bs, 3, 3, 24, nq), (0, 1, 5, 4, 2, 3))
    betas = jnp.transpose(betas, (0, 2, 3, 1))
    camera = jnp.transpose(cam, (0, 2, 3, 1))
    return (rotmat, betas, camera)


# speedup vs baseline: 1.2028x; 1.2028x over previous
import numpy as np
import jax
import jax.numpy as jnp
from jax.experimental import pallas as pl
from jax.experimental.pallas import tpu as pltpu

_RC = np.arange(9)
_J = np.arange(24)
_ROT_PERM = (_J[None, :] * 9 + _RC[:, None]).reshape(-1)


def _head_kernel(x_ref, wr_ref, wb_ref, wc_ref, rot_ref, betas_ref, cam_ref):
    blk = x_ref.shape[2]
    for b in range(blk):
        xb = x_ref[0, :, b, :]
        rot_ref[0, b] = jax.lax.dot_general(
            wr_ref[...], xb, (((1,), (1,)), ((), ())),
            preferred_element_type=jnp.float32)
        betas_ref[0, :, b, :] = jax.lax.dot_general(
            wb_ref[...], xb, (((1,), (1,)), ((), ())),
            preferred_element_type=jnp.float32)
        cam_ref[0, :, b, :] = jax.lax.dot_general(
            wc_ref[...], xb, (((1,), (1,)), ((), ())),
            preferred_element_type=jnp.float32)


def kernel(x, pred_class, W_rot, W_betas, W_cam):
    stage, bs, nq, ch = x.shape

    xv = jnp.transpose(x, (0, 2, 1, 3))
    wr_t = W_rot.T[_ROT_PERM]
    wb_t = W_betas.T
    wc_t = W_cam.T

    blk = 8
    grid = (stage, bs // blk)

    rot, betas, cam = pl.pallas_call(
        _head_kernel,
        grid=grid,
        in_specs=[
            pl.BlockSpec((1, nq, blk, ch), lambda i, j: (i, 0, j, 0)),
            pl.BlockSpec((216, ch), lambda i, j: (0, 0)),
            pl.BlockSpec((10, ch), lambda i, j: (0, 0)),
            pl.BlockSpec((3, ch), lambda i, j: (0, 0)),
        ],
        out_specs=[
            pl.BlockSpec((1, blk, 216, nq), lambda i, j: (i, j, 0, 0)),
            pl.BlockSpec((1, 10, blk, nq), lambda i, j: (i, 0, j, 0)),
            pl.BlockSpec((1, 3, blk, nq), lambda i, j: (i, 0, j, 0)),
        ],
        out_shape=[
            jax.ShapeDtypeStruct((stage, bs, 216, nq), jnp.float32),
            jax.ShapeDtypeStruct((stage, 10, bs, nq), jnp.float32),
            jax.ShapeDtypeStruct((stage, 3, bs, nq), jnp.float32),
        ],
        compiler_params=pltpu.CompilerParams(
            dimension_semantics=("parallel", "parallel")),
    )(xv, wr_t, wb_t, wc_t)

    rotmat = jnp.transpose(
        rot.reshape(stage, bs, 3, 3, 24, nq), (0, 1, 5, 4, 2, 3))
    betas = jnp.transpose(betas, (0, 2, 3, 1))
    camera = jnp.transpose(cam, (0, 2, 3, 1))
    return (rotmat, betas, camera)
